# baseline (speedup 1.0000x reference)
import functools

import jax
import jax.numpy as jnp
from jax import lax
from jax.experimental import pallas as pl
from jax.experimental.pallas import tpu as pltpu
from jax.experimental.pallas import tpu_sc as plsc

BATCH = 16384
DIM = 128
CHUNK = 128
NBUF = 7


@functools.lru_cache(maxsize=None)
def _make_kernel():
    info = plsc.get_sparse_core_info()
    nc = info.num_cores
    nw = nc * info.num_subcores
    b_per_w = BATCH // nw
    n_chunks = b_per_w // CHUNK

    mesh = plsc.VectorSubcoreMesh(core_axis_name="c", subcore_axis_name="s")

    @functools.partial(
        pl.kernel,
        mesh=mesh,
        out_type=(
            jax.ShapeDtypeStruct((BATCH, DIM), jnp.float32),
            jax.ShapeDtypeStruct((BATCH, DIM), jnp.float32),
        ),
        scratch_types=[
            pltpu.VMEM((n_chunks, CHUNK), jnp.int32),
            pltpu.VMEM((n_chunks, CHUNK), jnp.int32),
            pltpu.VMEM((NBUF * CHUNK, DIM), jnp.float32),
            pltpu.SemaphoreType.DMA,
            pltpu.SemaphoreType.DMA,
        ],
    )
    def k(uids, iids, utab, itab, uout, iout, uidx, iidx, rows, gsem, wsem):
        wid = lax.axis_index("s") * nc + lax.axis_index("c")
        base = wid * n_chunks
        c1 = pltpu.async_copy(uids.at[pl.ds(base, n_chunks)], uidx, wsem)
        c2 = pltpu.async_copy(iids.at[pl.ds(base, n_chunks)], iidx, wsem)
        c1.wait()
        c2.wait()

        tasks = []
        for j in range(n_chunks):
            tasks.append((uidx.at[j], utab, uout, wid * b_per_w + j * CHUNK))
            tasks.append((iidx.at[j], itab, iout, wid * b_per_w + j * CHUNK))
        nt = len(tasks)
        LAG = 1
        gathers = [None] * nt
        writes = [None] * nt

        def buf(t):
            return rows.at[pl.ds((t % NBUF) * CHUNK, CHUNK)]

        for t in range(nt + LAG):
            if t < nt:
                if t >= NBUF:
                    writes[t - NBUF].wait()
                idx_row, tab, _, _ = tasks[t]
                gathers[t] = pltpu.async_copy(tab.at[idx_row], buf(t), gsem)
            if t >= LAG:
                s = t - LAG
                gathers[s].wait()
                _, _, out, off = tasks[s]
                writes[s] = pltpu.async_copy(
                    buf(s), out.at[pl.ds(off, CHUNK)], wsem
                )
        for s in range(nt - NBUF, nt):
            writes[s].wait()

    return k


def kernel(user_ids, item_ids, user_table, item_table):
    uids = user_ids.astype(jnp.int32).reshape(BATCH // CHUNK, CHUNK)
    iids = item_ids.astype(jnp.int32).reshape(BATCH // CHUNK, CHUNK)
    return _make_kernel()(uids, iids, user_table, item_table)


# speedup vs baseline: 1.0034x; 1.0034x over previous
import functools

import jax
import jax.numpy as jnp
from jax import lax
from jax.experimental import pallas as pl
from jax.experimental.pallas import tpu as pltpu
from jax.experimental.pallas import tpu_sc as plsc

BATCH = 16384
DIM = 128
CHUNK = 128
NBUF = 7


@functools.lru_cache(maxsize=None)
def _make_kernel():
    info = plsc.get_sparse_core_info()
    nc = info.num_cores
    nw = nc * info.num_subcores
    b_per_w = BATCH // nw
    n_chunks = b_per_w // CHUNK

    mesh = plsc.VectorSubcoreMesh(core_axis_name="c", subcore_axis_name="s")

    @functools.partial(
        pl.kernel,
        mesh=mesh,
        out_type=(
            jax.ShapeDtypeStruct((BATCH, DIM), jnp.float32),
            jax.ShapeDtypeStruct((BATCH, DIM), jnp.float32),
        ),
        scratch_types=[
            pltpu.VMEM((n_chunks, CHUNK), jnp.int32),
            pltpu.VMEM((n_chunks, CHUNK), jnp.int32),
            pltpu.VMEM((NBUF * CHUNK, DIM), jnp.float32),
            pltpu.SemaphoreType.DMA,
            pltpu.SemaphoreType.DMA,
        ],
    )
    def k(uids, iids, utab, itab, uout, iout, uidx, iidx, rows, gsem, wsem):
        wid = lax.axis_index("s") * nc + lax.axis_index("c")
        base = wid * n_chunks
        c1 = pltpu.async_copy(uids.at[pl.ds(base, n_chunks)], uidx, wsem)
        c2 = pltpu.async_copy(iids.at[pl.ds(base, n_chunks)], iidx, wsem)

        tasks = []
        for j in range(n_chunks):
            tasks.append((uidx.at[j], utab, uout, wid * b_per_w + j * CHUNK))
            tasks.append((iidx.at[j], itab, iout, wid * b_per_w + j * CHUNK))
        nt = len(tasks)
        LAG = 1
        gathers = [None] * nt
        writes = [None] * nt

        def buf(t):
            return rows.at[pl.ds((t % NBUF) * CHUNK, CHUNK)]

        for t in range(nt + LAG):
            if t < nt:
                if t == 0:
                    c1.wait()
                if t == 1:
                    c2.wait()
                if t >= NBUF:
                    writes[t - NBUF].wait()
                idx_row, tab, _, _ = tasks[t]
                gathers[t] = pltpu.async_copy(tab.at[idx_row], buf(t), gsem)
            if t >= LAG:
                s = t - LAG
                gathers[s].wait()
                _, _, out, off = tasks[s]
                writes[s] = pltpu.async_copy(
                    buf(s), out.at[pl.ds(off, CHUNK)], wsem
                )
        for s in range(nt - NBUF, nt):
            writes[s].wait()

    return k


def kernel(user_ids, item_ids, user_table, item_table):
    uids = user_ids.astype(jnp.int32).reshape(BATCH // CHUNK, CHUNK)
    iids = item_ids.astype(jnp.int32).reshape(BATCH // CHUNK, CHUNK)
    return _make_kernel()(uids, iids, user_table, item_table)
